# baseline (speedup 1.0000x reference)
import functools

import jax
import jax.numpy as jnp
from jax import lax
from jax.experimental import pallas as pl
from jax.experimental.pallas import tpu as pltpu
from jax.experimental.pallas import tpu_sc as plsc

B = 16384
D = 32

_info = plsc.get_sparse_core_info()
_NC, _NS = _info.num_cores, _info.num_subcores
_NW = _NC * _NS
_BPW = B // _NW
_CHUNK = 128
_NCHUNK = _BPW // _CHUNK


def _sc_gather(uid, iid, ug_t, ig_t, um_t, im_t):
    mesh = plsc.VectorSubcoreMesh(core_axis_name="c", subcore_axis_name="s")
    out_t = [jax.ShapeDtypeStruct((B, D), jnp.float32)] * 4

    @functools.partial(
        pl.kernel,
        mesh=mesh,
        out_type=out_t,
        compiler_params=pltpu.CompilerParams(use_tc_tiling_on_sc=False),
        scratch_types=[
            pltpu.VMEM((_BPW,), jnp.int32),
            pltpu.VMEM((_BPW,), jnp.int32),
            pltpu.VMEM((_BPW, D), jnp.float32),
            pltpu.VMEM((_BPW, D), jnp.float32),
            pltpu.VMEM((_BPW, D), jnp.float32),
            pltpu.VMEM((_BPW, D), jnp.float32),
            pltpu.SemaphoreType.DMA,
        ],
    )
    def k(uid_h, iid_h, ugt_h, igt_h, umt_h, imt_h,
          oug_h, oig_h, oum_h, oim_h,
          idx_u, idx_i, bug, big, bum, bim, sem):
        wid = lax.axis_index("s") * _NC + lax.axis_index("c")
        base = wid * _BPW
        pltpu.sync_copy(uid_h.at[pl.ds(base, _BPW)], idx_u)
        pltpu.sync_copy(iid_h.at[pl.ds(base, _BPW)], idx_i)
        cps = []
        for c in range(_NCHUNK):
            s = pl.ds(c * _CHUNK, _CHUNK)
            cps.append(pltpu.async_copy(ugt_h.at[idx_u.at[s]], bug.at[s], sem))
            cps.append(pltpu.async_copy(igt_h.at[idx_i.at[s]], big.at[s], sem))
            cps.append(pltpu.async_copy(umt_h.at[idx_u.at[s]], bum.at[s], sem))
            cps.append(pltpu.async_copy(imt_h.at[idx_i.at[s]], bim.at[s], sem))
        for cp in cps:
            cp.wait()
        dst = pl.ds(base, _BPW)
        pltpu.sync_copy(bug, oug_h.at[dst])
        pltpu.sync_copy(big, oig_h.at[dst])
        pltpu.sync_copy(bum, oum_h.at[dst])
        pltpu.sync_copy(bim, oim_h.at[dst])

    return k(uid, iid, ug_t, ig_t, um_t, im_t)


_BLK = 2048


def _mlp_body(ug, ig, um, im, w1a, w1b, b1, w2, b2, w3, b3, wpg, wph, bp, out):
    h = jnp.maximum(
        jnp.dot(um[...], w1a[...], preferred_element_type=jnp.float32)
        + jnp.dot(im[...], w1b[...], preferred_element_type=jnp.float32)
        + b1[...], 0.0)
    h = jnp.maximum(
        jnp.dot(h, w2[...], preferred_element_type=jnp.float32) + b2[...], 0.0)
    h = jnp.maximum(
        jnp.dot(h, w3[...], preferred_element_type=jnp.float32) + b3[...], 0.0)
    g = ug[...] * ig[...]
    logit = (jnp.sum(g * wpg[...], axis=1)
             + jnp.sum(h * wph[...], axis=1) + bp[0, 0])
    out[...] = jax.nn.sigmoid(logit)


def _tc_mlp(ug, ig, um, im, W1, b1, W2, b2, W3, b3, Wp, bp):
    w1a, w1b = W1[:D], W1[D:]
    wpg = Wp[:D, 0].reshape(1, D)
    wph = Wp[D:, 0].reshape(1, D)
    b1r = b1.reshape(1, -1)
    b2r = b2.reshape(1, -1)
    b3r = b3.reshape(1, -1)
    bpr = bp.reshape(1, 1)

    grid = B // _BLK
    row_spec = pl.BlockSpec((_BLK, D), lambda i: (i, 0))
    full = lambda a: pl.BlockSpec(a.shape, lambda i: (0,) * a.ndim)
    return pl.pallas_call(
        _mlp_body,
        grid=(grid,),
        in_specs=[
            row_spec, row_spec, row_spec, row_spec,
            full(w1a), full(w1b), full(b1r),
            full(W2), full(b2r), full(W3), full(b3r),
            full(wpg), full(wph),
            pl.BlockSpec(memory_space=pltpu.SMEM),
        ],
        out_specs=pl.BlockSpec((_BLK,), lambda i: (i,)),
        out_shape=jax.ShapeDtypeStruct((B,), jnp.float32),
    )(ug, ig, um, im, w1a, w1b, b1r, W2, b2r, W3, b3r, wpg, wph, bpr)


def kernel(user_ids, item_ids, user_emb_gmf, item_emb_gmf, user_emb_mlp,
           item_emb_mlp, W1, b1, W2, b2, W3, b3, Wp, bp):
    ug, ig, um, im = _sc_gather(user_ids, item_ids, user_emb_gmf,
                                item_emb_gmf, user_emb_mlp, item_emb_mlp)
    return _tc_mlp(ug, ig, um, im, W1, b1, W2, b2, W3, b3, Wp, bp)


# speedup vs baseline: 1.0004x; 1.0004x over previous
import functools

import jax
import jax.numpy as jnp
from jax import lax
from jax.experimental import pallas as pl
from jax.experimental.pallas import tpu as pltpu
from jax.experimental.pallas import tpu_sc as plsc

B = 16384
D = 32

_info = plsc.get_sparse_core_info()
_NC, _NS = _info.num_cores, _info.num_subcores
_NW = _NC * _NS
_BPW = B // _NW


def _sc_gather(uid, iid, ug_t, ig_t, um_t, im_t):
    mesh = plsc.VectorSubcoreMesh(core_axis_name="c", subcore_axis_name="s")
    out_t = [jax.ShapeDtypeStruct((B, D), jnp.float32)] * 4

    @functools.partial(
        pl.kernel,
        mesh=mesh,
        out_type=out_t,
        compiler_params=pltpu.CompilerParams(use_tc_tiling_on_sc=False),
        scratch_types=[
            pltpu.VMEM((_BPW,), jnp.int32),
            pltpu.VMEM((_BPW,), jnp.int32),
            pltpu.VMEM((_BPW, D), jnp.float32),
            pltpu.VMEM((_BPW, D), jnp.float32),
            pltpu.VMEM((_BPW, D), jnp.float32),
            pltpu.VMEM((_BPW, D), jnp.float32),
        ],
    )
    def k(uid_h, iid_h, ugt_h, igt_h, umt_h, imt_h,
          oug_h, oig_h, oum_h, oim_h,
          idx_u, idx_i, b0, b1, b2, b3):
        wid = lax.axis_index("s") * _NC + lax.axis_index("c")
        sl = pl.ds(wid * _BPW, _BPW)
        pltpu.sync_copy(uid_h.at[sl], idx_u)
        pltpu.sync_copy(iid_h.at[sl], idx_i)

        def body(gsem, ssem):
            gathers = (
                pltpu.async_copy(ugt_h.at[idx_u], b0, gsem),
                pltpu.async_copy(igt_h.at[idx_i], b1, gsem),
                pltpu.async_copy(umt_h.at[idx_u], b2, gsem),
                pltpu.async_copy(imt_h.at[idx_i], b3, gsem),
            )
            stores = []
            for g, buf, out in zip(gathers, (b0, b1, b2, b3),
                                   (oug_h, oig_h, oum_h, oim_h)):
                g.wait()
                stores.append(pltpu.async_copy(buf, out.at[sl], ssem))
            for s in stores:
                s.wait()

        pl.run_scoped(body, pltpu.SemaphoreType.DMA(()),
                      pltpu.SemaphoreType.DMA(()))

    return k(uid, iid, ug_t, ig_t, um_t, im_t)


_BLK = 2048


def _mlp_body(ug, ig, um, im, w1a, w1b, b1, w2, b2, w3, b3, wpg, wph, bp, out):
    h = jnp.maximum(
        jnp.dot(um[...], w1a[...], preferred_element_type=jnp.float32)
        + jnp.dot(im[...], w1b[...], preferred_element_type=jnp.float32)
        + b1[...], 0.0)
    h = jnp.maximum(
        jnp.dot(h, w2[...], preferred_element_type=jnp.float32) + b2[...], 0.0)
    h = jnp.maximum(
        jnp.dot(h, w3[...], preferred_element_type=jnp.float32) + b3[...], 0.0)
    g = ug[...] * ig[...]
    logit = (jnp.sum(g * wpg[...], axis=1)
             + jnp.sum(h * wph[...], axis=1) + bp[0, 0])
    out[...] = jax.nn.sigmoid(logit)


def _tc_mlp(ug, ig, um, im, W1, b1, W2, b2, W3, b3, Wp, bp):
    w1a, w1b = W1[:D], W1[D:]
    wpg = Wp[:D, 0].reshape(1, D)
    wph = Wp[D:, 0].reshape(1, D)
    b1r = b1.reshape(1, -1)
    b2r = b2.reshape(1, -1)
    b3r = b3.reshape(1, -1)
    bpr = bp.reshape(1, 1)

    grid = B // _BLK
    row_spec = pl.BlockSpec((_BLK, D), lambda i: (i, 0))
    full = lambda a: pl.BlockSpec(a.shape, lambda i: (0,) * a.ndim)
    return pl.pallas_call(
        _mlp_body,
        grid=(grid,),
        in_specs=[
            row_spec, row_spec, row_spec, row_spec,
            full(w1a), full(w1b), full(b1r),
            full(W2), full(b2r), full(W3), full(b3r),
            full(wpg), full(wph),
            pl.BlockSpec(memory_space=pltpu.SMEM),
        ],
        out_specs=pl.BlockSpec((_BLK,), lambda i: (i,)),
        out_shape=jax.ShapeDtypeStruct((B,), jnp.float32),
    )(ug, ig, um, im, w1a, w1b, b1r, W2, b2r, W3, b3r, wpg, wph, bpr)


def kernel(user_ids, item_ids, user_emb_gmf, item_emb_gmf, user_emb_mlp,
           item_emb_mlp, W1, b1, W2, b2, W3, b3, Wp, bp):
    ug, ig, um, im = _sc_gather(user_ids, item_ids, user_emb_gmf,
                                item_emb_gmf, user_emb_mlp, item_emb_mlp)
    return _tc_mlp(ug, ig, um, im, W1, b1, W2, b2, W3, b3, Wp, bp)
